# baseline (speedup 1.0000x reference)
import functools

import jax
import jax.numpy as jnp
from jax import lax
from jax.experimental import pallas as pl
from jax.experimental.pallas import tpu as pltpu
from jax.experimental.pallas import tpu_sc as plsc

_B, _C, _H, _W = 8, 96, 224, 224
_E = 64
_HW = _H * _W
_OUTB = (_C + _E) * _HW
_IMGB = _C * _HW
_NW = 32
_EPW = _E * _B // _NW
_IPW = _C * _B // _NW
_FILL_UNROLL = 8
_FILL_ITERS = _HW // (16 * _FILL_UNROLL)


def _sc_body(idx_hbm, emb_hbm, img_hbm, out_hbm,
             idx_v, rows_v, buf0, buf1, sem_g, sem_in, sem_out):
    w = lax.axis_index("s") * 2 + lax.axis_index("c")
    bufs = (buf0, buf1)

    pltpu.sync_copy(idx_hbm, idx_v)
    pltpu.async_copy(emb_hbm.at[idx_v], rows_v, sem_g).wait()

    b_e = w // 4
    c0 = (w % 4) * _EPW
    vals16 = rows_v[b_e, pl.ds(c0, _EPW)]

    emb_copies = []
    for j in range(_EPW):
        buf = bufs[j % 2]
        if j >= 2:
            emb_copies[j - 2].wait()
        splat = jnp.full((16,), vals16[j], jnp.float32)

        def _fill(i, _, buf=buf, splat=splat):
            base = i * (16 * _FILL_UNROLL)
            for u in range(_FILL_UNROLL):
                buf[pl.ds(base + u * 16, 16)] = splat
            return _

        lax.fori_loop(0, _FILL_ITERS, _fill, None)
        off = pl.multiple_of(b_e * _OUTB + (c0 + j) * _HW, 8)
        cp = pltpu.make_async_copy(buf, out_hbm.at[pl.ds(off, _HW)], sem_out)
        cp.start()
        emb_copies.append(cp)
    emb_copies[_EPW - 2].wait()
    emb_copies[_EPW - 1].wait()

    def _img_in(t, buf):
        k = w * _IPW + t
        b = k // _C
        c = k % _C
        src = pl.multiple_of(b * _IMGB + c * _HW, 8)
        return pltpu.make_async_copy(img_hbm.at[pl.ds(src, _HW)], buf, sem_in)

    def _img_out(t, buf):
        k = w * _IPW + t
        b = k // _C
        c = k % _C
        dst = pl.multiple_of(b * _OUTB + _E * _HW + c * _HW, 8)
        return pltpu.make_async_copy(buf, out_hbm.at[pl.ds(dst, _HW)], sem_out)

    in_cp = [None] * _IPW
    out_cp = [None] * _IPW
    in_cp[0] = _img_in(0, bufs[0])
    in_cp[0].start()
    for t in range(_IPW):
        buf = bufs[t % 2]
        in_cp[t].wait()
        if t >= 1:
            out_cp[t - 1].wait()
        if t + 1 < _IPW:
            in_cp[t + 1] = _img_in(t + 1, bufs[(t + 1) % 2])
            in_cp[t + 1].start()
        out_cp[t] = _img_out(t, buf)
        out_cp[t].start()
    out_cp[_IPW - 1].wait()


def kernel(class_idx, image, emb_table):
    idx16 = jnp.pad(class_idx.astype(jnp.int32), (0, 16 - _B))
    img_flat = image.reshape(-1)

    k = functools.partial(
        pl.kernel,
        mesh=plsc.VectorSubcoreMesh(core_axis_name="c", subcore_axis_name="s"),
        out_type=jax.ShapeDtypeStruct((_B * _OUTB,), jnp.float32),
        scratch_types=[
            pltpu.VMEM((16,), jnp.int32),
            pltpu.VMEM((16, 128), jnp.float32),
            pltpu.VMEM((_HW,), jnp.float32),
            pltpu.VMEM((_HW,), jnp.float32),
            pltpu.SemaphoreType.DMA,
            pltpu.SemaphoreType.DMA,
            pltpu.SemaphoreType.DMA,
        ],
    )(_sc_body)
    table128 = jnp.pad(emb_table, ((0, 0), (0, 128 - _E)))
    out_flat = k(idx16, table128, img_flat)
    return out_flat.reshape(_B, _C + _E, _H, _W)


# speedup vs baseline: 4.8304x; 4.8304x over previous
import jax
import jax.numpy as jnp
from jax.experimental import pallas as pl
from jax.experimental.pallas import tpu as pltpu

_B, _C, _H, _W = 8, 96, 224, 224
_E = 64
_HB = 112


def _body(idx_ref, emb_row_ref, img_ref, out_ref):
    row = emb_row_ref[0, 0, :]
    out_ref[0, :_E] = jnp.broadcast_to(row[:, None, None], (_E, _HB, _W))
    out_ref[0, _E:] = img_ref[0]


def kernel(class_idx, image, emb_table):
    grid = (_B, _H // _HB)
    return pl.pallas_call(
        _body,
        grid_spec=pltpu.PrefetchScalarGridSpec(
            num_scalar_prefetch=1,
            grid=grid,
            in_specs=[
                pl.BlockSpec((1, 1, _E), lambda b, h, idx_ref: (idx_ref[b], 0, 0)),
                pl.BlockSpec((1, _C, _HB, _W), lambda b, h, idx_ref: (b, 0, h, 0)),
            ],
            out_specs=pl.BlockSpec((1, _C + _E, _HB, _W),
                                   lambda b, h, idx_ref: (b, 0, h, 0)),
        ),
        out_shape=jax.ShapeDtypeStruct((_B, _C + _E, _H, _W), jnp.float32),
    )(class_idx, emb_table.reshape(-1, 1, _E), image)
